# baseline (speedup 1.0000x reference)
import jax
import jax.numpy as jnp
from jax import lax
from jax.experimental import pallas as pl
from jax.experimental.pallas import tpu as pltpu
from jax.experimental.pallas import tpu_sc as plsc

NUM_FIELDS = 26
BATCH = 16384
VOCAB = 100000
EMB = 16
H1 = 128
H2 = 64
NW = 32
BPW = BATCH // NW


def _gather_body(*refs):
    tables = refs[:NUM_FIELDS]
    idxs = refs[NUM_FIELDS:2 * NUM_FIELDS]
    out = refs[2 * NUM_FIELDS]
    idx_v, rows_v, sem = refs[2 * NUM_FIELDS + 1:]
    wid = lax.axis_index("s") * 2 + lax.axis_index("c")
    base = wid * BPW
    for f in range(NUM_FIELDS):
        pltpu.sync_copy(idxs[f].at[pl.ds(base, BPW)], idx_v)
        pltpu.async_copy(tables[f].at[idx_v], rows_v, sem).wait()
        pltpu.sync_copy(rows_v, out.at[pl.ds(base, BPW), pl.ds(f * EMB, EMB)])


_gather = pl.kernel(
    _gather_body,
    out_type=jax.ShapeDtypeStruct((BATCH, NUM_FIELDS * EMB), jnp.float32),
    mesh=plsc.VectorSubcoreMesh(core_axis_name="c", subcore_axis_name="s"),
    scratch_types=[
        pltpu.VMEM((BPW,), jnp.int32),
        pltpu.VMEM((BPW, EMB), jnp.float32),
        pltpu.SemaphoreType.DMA,
    ],
    compiler_params=pltpu.CompilerParams(use_tc_tiling_on_sc=False),
)


def _pack_body(*refs):
    ins = refs[:NUM_FIELDS]
    outs = refs[NUM_FIELDS:]
    eye = jnp.eye(EMB, dtype=jnp.float32)
    for t, o in zip(ins, outs):
        o[...] = jax.lax.dot_general(
            t[...], eye, (((0,), (0,)), ((), ())),
            preferred_element_type=jnp.float32)


def _pack(tabTs, bn=512):
    nb = -(-VOCAB // bn)
    return pl.pallas_call(
        _pack_body,
        grid=(nb,),
        in_specs=[pl.BlockSpec((EMB, bn), lambda i: (0, i))] * NUM_FIELDS,
        out_specs=[pl.BlockSpec((bn, EMB), lambda i: (i, 0))] * NUM_FIELDS,
        out_shape=[jax.ShapeDtypeStruct((VOCAB, EMB), jnp.float32)] * NUM_FIELDS,
    )(*tabTs)


def _mlp_body(x_ref, w1_ref, b1_ref, w2_ref, b2_ref, o_ref):
    h = jnp.dot(x_ref[...], w1_ref[...], preferred_element_type=jnp.float32)
    h = jnp.maximum(h + b1_ref[...], 0.0)
    o = jnp.dot(h, w2_ref[...], preferred_element_type=jnp.float32)
    o_ref[...] = jnp.maximum(o + b2_ref[...], 0.0)


def _mlp(x, w1, b1, w2, b2, bb=2048):
    d = NUM_FIELDS * EMB
    return pl.pallas_call(
        _mlp_body,
        grid=(BATCH // bb,),
        in_specs=[
            pl.BlockSpec((bb, d), lambda i: (i, 0)),
            pl.BlockSpec((d, H1), lambda i: (0, 0)),
            pl.BlockSpec((1, H1), lambda i: (0, 0)),
            pl.BlockSpec((H1, H2), lambda i: (0, 0)),
            pl.BlockSpec((1, H2), lambda i: (0, 0)),
        ],
        out_specs=pl.BlockSpec((bb, H2), lambda i: (i, 0)),
        out_shape=jax.ShapeDtypeStruct((BATCH, H2), jnp.float32),
    )(x, w1, b1, w2, b2)


def kernel(feat_0, feat_1, feat_2, feat_3, feat_4, feat_5, feat_6, feat_7,
           feat_8, feat_9, feat_10, feat_11, feat_12, feat_13, feat_14,
           feat_15, feat_16, feat_17, feat_18, feat_19, feat_20, feat_21,
           feat_22, feat_23, feat_24, feat_25,
           E_0, E_1, E_2, E_3, E_4, E_5, E_6, E_7, E_8, E_9, E_10, E_11,
           E_12, E_13, E_14, E_15, E_16, E_17, E_18, E_19, E_20, E_21,
           E_22, E_23, E_24, E_25,
           W1, b1, W2, b2):
    feats = [feat_0, feat_1, feat_2, feat_3, feat_4, feat_5, feat_6, feat_7,
             feat_8, feat_9, feat_10, feat_11, feat_12, feat_13, feat_14,
             feat_15, feat_16, feat_17, feat_18, feat_19, feat_20, feat_21,
             feat_22, feat_23, feat_24, feat_25]
    tables = [E_0, E_1, E_2, E_3, E_4, E_5, E_6, E_7, E_8, E_9, E_10, E_11,
              E_12, E_13, E_14, E_15, E_16, E_17, E_18, E_19, E_20, E_21,
              E_22, E_23, E_24, E_25]
    feats = [jnp.asarray(f, jnp.int32) for f in feats]
    tables = _pack([t.T for t in tables])
    x = _gather(*tables, *feats)
    return _mlp(x, W1, b1.reshape(1, H1), W2, b2.reshape(1, H2))


# speedup vs baseline: 1.7359x; 1.7359x over previous
import jax
import jax.numpy as jnp
from jax import lax
from jax.experimental import pallas as pl
from jax.experimental.pallas import tpu as pltpu
from jax.experimental.pallas import tpu_sc as plsc

NUM_FIELDS = 26
BATCH = 16384
VOCAB = 100000
VOCAB_P = 100352
EMB = 16
H1 = 128
H2 = 64
NW = 32
BPW = BATCH // NW

PACK_BN = 1024
PACK_GRID = -(-VOCAB_P // PACK_BN)


def _pack_body(*refs):
    ins = refs[:NUM_FIELDS]
    outs = refs[NUM_FIELDS:]
    for t, o in zip(ins, outs):
        x = t[...]
        o[...] = jnp.concatenate(
            [x[:, 128 * k:128 * (k + 1)].T for k in range(8)], axis=1)


def _pack(tabTs):
    return pl.pallas_call(
        _pack_body,
        grid=(PACK_GRID,),
        in_specs=[pl.BlockSpec((EMB, PACK_BN), lambda i: (0, i))] * NUM_FIELDS,
        out_specs=[pl.BlockSpec((PACK_BN // 8, 8 * EMB), lambda i: (i, 0))] * NUM_FIELDS,
        out_shape=[jax.ShapeDtypeStruct((VOCAB_P // 8, 8 * EMB), jnp.float32)] * NUM_FIELDS,
    )(*tabTs)


def _gather_body(*refs):
    tables = refs[:NUM_FIELDS]
    idxs = refs[NUM_FIELDS:2 * NUM_FIELDS]
    out = refs[2 * NUM_FIELDS]
    idx_v, rows_v, sem = refs[2 * NUM_FIELDS + 1:]
    wid = lax.axis_index("s") * 2 + lax.axis_index("c")
    base = wid * BPW
    for f in range(NUM_FIELDS):
        pltpu.sync_copy(idxs[f].at[pl.ds(base, BPW)], idx_v)
        pltpu.async_copy(tables[f].at[idx_v], rows_v, sem).wait()
        pltpu.sync_copy(rows_v, out.at[pl.ds(base, BPW), pl.ds(f * EMB, EMB)])


_gather = pl.kernel(
    _gather_body,
    out_type=jax.ShapeDtypeStruct((BATCH, NUM_FIELDS * EMB), jnp.float32),
    mesh=plsc.VectorSubcoreMesh(core_axis_name="c", subcore_axis_name="s"),
    scratch_types=[
        pltpu.VMEM((BPW,), jnp.int32),
        pltpu.VMEM((BPW, EMB), jnp.float32),
        pltpu.SemaphoreType.DMA,
    ],
    compiler_params=pltpu.CompilerParams(use_tc_tiling_on_sc=False),
)


def _mlp_body(x_ref, w1_ref, b1_ref, w2_ref, b2_ref, o_ref):
    h = jnp.dot(x_ref[...], w1_ref[...], preferred_element_type=jnp.float32)
    h = jnp.maximum(h + b1_ref[...], 0.0)
    o = jnp.dot(h, w2_ref[...], preferred_element_type=jnp.float32)
    o_ref[...] = jnp.maximum(o + b2_ref[...], 0.0)


def _mlp(x, w1, b1, w2, b2, bb=2048):
    d = NUM_FIELDS * EMB
    return pl.pallas_call(
        _mlp_body,
        grid=(BATCH // bb,),
        in_specs=[
            pl.BlockSpec((bb, d), lambda i: (i, 0)),
            pl.BlockSpec((d, H1), lambda i: (0, 0)),
            pl.BlockSpec((1, H1), lambda i: (0, 0)),
            pl.BlockSpec((H1, H2), lambda i: (0, 0)),
            pl.BlockSpec((1, H2), lambda i: (0, 0)),
        ],
        out_specs=pl.BlockSpec((bb, H2), lambda i: (i, 0)),
        out_shape=jax.ShapeDtypeStruct((BATCH, H2), jnp.float32),
    )(x, w1, b1, w2, b2)


def kernel(feat_0, feat_1, feat_2, feat_3, feat_4, feat_5, feat_6, feat_7,
           feat_8, feat_9, feat_10, feat_11, feat_12, feat_13, feat_14,
           feat_15, feat_16, feat_17, feat_18, feat_19, feat_20, feat_21,
           feat_22, feat_23, feat_24, feat_25,
           E_0, E_1, E_2, E_3, E_4, E_5, E_6, E_7, E_8, E_9, E_10, E_11,
           E_12, E_13, E_14, E_15, E_16, E_17, E_18, E_19, E_20, E_21,
           E_22, E_23, E_24, E_25,
           W1, b1, W2, b2):
    feats = [feat_0, feat_1, feat_2, feat_3, feat_4, feat_5, feat_6, feat_7,
             feat_8, feat_9, feat_10, feat_11, feat_12, feat_13, feat_14,
             feat_15, feat_16, feat_17, feat_18, feat_19, feat_20, feat_21,
             feat_22, feat_23, feat_24, feat_25]
    tables = [E_0, E_1, E_2, E_3, E_4, E_5, E_6, E_7, E_8, E_9, E_10, E_11,
              E_12, E_13, E_14, E_15, E_16, E_17, E_18, E_19, E_20, E_21,
              E_22, E_23, E_24, E_25]
    feats = [jnp.asarray(f, jnp.int32) for f in feats]
    feats = [
        (f & ~1023) + ((f & 127) << 3) + ((f >> 7) & 7) for f in feats
    ]
    packed = _pack([t.T for t in tables])
    tables = [p.reshape(VOCAB_P, EMB) for p in packed]
    x = _gather(*tables, *feats)
    return _mlp(x, W1, b1.reshape(1, H1), W2, b2.reshape(1, H2))


# speedup vs baseline: 1.7752x; 1.0226x over previous
import jax
import jax.numpy as jnp
from jax import lax
from jax.experimental import pallas as pl
from jax.experimental.pallas import tpu as pltpu
from jax.experimental.pallas import tpu_sc as plsc

NUM_FIELDS = 26
BATCH = 16384
VOCAB = 100000
VOCAB_P = 100352
EMB = 16
H1 = 128
H2 = 64
NW = 32
BPW = BATCH // NW

PACK_BN = 2048
PACK_GRID = -(-VOCAB_P // PACK_BN)


def _pack_body(*refs):
    ins = refs[:NUM_FIELDS]
    outs = refs[NUM_FIELDS:]
    eye = jnp.eye(EMB, dtype=jnp.float32)
    for t, o in zip(ins, outs):
        x = t[...]
        for h in range(PACK_BN // 1024):
            o[128 * h:128 * (h + 1), :] = jnp.concatenate(
                [jax.lax.dot_general(
                    x[:, 128 * (8 * h + k):128 * (8 * h + k + 1)], eye,
                    (((0,), (0,)), ((), ())),
                    preferred_element_type=jnp.float32) for k in range(8)],
                axis=1)


def _pack(tabTs):
    return pl.pallas_call(
        _pack_body,
        grid=(PACK_GRID,),
        in_specs=[pl.BlockSpec((EMB, PACK_BN), lambda i: (0, i))] * NUM_FIELDS,
        out_specs=[pl.BlockSpec((PACK_BN // 8, 8 * EMB), lambda i: (i, 0))] * NUM_FIELDS,
        out_shape=[jax.ShapeDtypeStruct((VOCAB_P // 8, 8 * EMB), jnp.float32)] * NUM_FIELDS,
    )(*tabTs)


def _gather_body(*refs):
    tables = refs[:NUM_FIELDS]
    idxs = refs[NUM_FIELDS:2 * NUM_FIELDS]
    out = refs[2 * NUM_FIELDS]
    idx_v, rows_v, sem = refs[2 * NUM_FIELDS + 1:]
    wid = lax.axis_index("s") * 2 + lax.axis_index("c")
    base = wid * BPW
    for f in range(NUM_FIELDS):
        pltpu.sync_copy(idxs[f].at[pl.ds(base, BPW)], idx_v)
        pltpu.async_copy(tables[f].at[idx_v], rows_v, sem).wait()
        pltpu.sync_copy(rows_v, out.at[pl.ds(base, BPW), pl.ds(f * EMB, EMB)])


_gather = pl.kernel(
    _gather_body,
    out_type=jax.ShapeDtypeStruct((BATCH, NUM_FIELDS * EMB), jnp.float32),
    mesh=plsc.VectorSubcoreMesh(core_axis_name="c", subcore_axis_name="s"),
    scratch_types=[
        pltpu.VMEM((BPW,), jnp.int32),
        pltpu.VMEM((BPW, EMB), jnp.float32),
        pltpu.SemaphoreType.DMA,
    ],
    compiler_params=pltpu.CompilerParams(use_tc_tiling_on_sc=False),
)


def _mlp_body(x_ref, w1_ref, b1_ref, w2_ref, b2_ref, o_ref):
    h = jnp.dot(x_ref[...], w1_ref[...], preferred_element_type=jnp.float32)
    h = jnp.maximum(h + b1_ref[...], 0.0)
    o = jnp.dot(h, w2_ref[...], preferred_element_type=jnp.float32)
    o_ref[...] = jnp.maximum(o + b2_ref[...], 0.0)


def _mlp(x, w1, b1, w2, b2, bb=2048):
    d = NUM_FIELDS * EMB
    return pl.pallas_call(
        _mlp_body,
        grid=(BATCH // bb,),
        in_specs=[
            pl.BlockSpec((bb, d), lambda i: (i, 0)),
            pl.BlockSpec((d, H1), lambda i: (0, 0)),
            pl.BlockSpec((1, H1), lambda i: (0, 0)),
            pl.BlockSpec((H1, H2), lambda i: (0, 0)),
            pl.BlockSpec((1, H2), lambda i: (0, 0)),
        ],
        out_specs=pl.BlockSpec((bb, H2), lambda i: (i, 0)),
        out_shape=jax.ShapeDtypeStruct((BATCH, H2), jnp.float32),
    )(x, w1, b1, w2, b2)


def kernel(feat_0, feat_1, feat_2, feat_3, feat_4, feat_5, feat_6, feat_7,
           feat_8, feat_9, feat_10, feat_11, feat_12, feat_13, feat_14,
           feat_15, feat_16, feat_17, feat_18, feat_19, feat_20, feat_21,
           feat_22, feat_23, feat_24, feat_25,
           E_0, E_1, E_2, E_3, E_4, E_5, E_6, E_7, E_8, E_9, E_10, E_11,
           E_12, E_13, E_14, E_15, E_16, E_17, E_18, E_19, E_20, E_21,
           E_22, E_23, E_24, E_25,
           W1, b1, W2, b2):
    feats = [feat_0, feat_1, feat_2, feat_3, feat_4, feat_5, feat_6, feat_7,
             feat_8, feat_9, feat_10, feat_11, feat_12, feat_13, feat_14,
             feat_15, feat_16, feat_17, feat_18, feat_19, feat_20, feat_21,
             feat_22, feat_23, feat_24, feat_25]
    tables = [E_0, E_1, E_2, E_3, E_4, E_5, E_6, E_7, E_8, E_9, E_10, E_11,
              E_12, E_13, E_14, E_15, E_16, E_17, E_18, E_19, E_20, E_21,
              E_22, E_23, E_24, E_25]
    feats = [jnp.asarray(f, jnp.int32) for f in feats]
    feats = [
        (f & ~1023) + ((f & 127) << 3) + ((f >> 7) & 7) for f in feats
    ]
    packed = _pack([t.T for t in tables])
    tables = [p.reshape(VOCAB_P, EMB) for p in packed]
    x = _gather(*tables, *feats)
    return _mlp(x, W1, b1.reshape(1, H1), W2, b2.reshape(1, H2))


# speedup vs baseline: 5.1062x; 2.8764x over previous
import jax
import jax.numpy as jnp
from jax import lax
from jax.experimental import pallas as pl
from jax.experimental.pallas import tpu as pltpu
from jax.experimental.pallas import tpu_sc as plsc

NUM_FIELDS = 26
BATCH = 16384
VOCAB = 100000
VOCAB_P = 100352
EMB = 16
H1 = 128
H2 = 64
NW = 32
BPW = BATCH // NW

PACK_BN = 2048
PACK_GRID = -(-VOCAB_P // PACK_BN)


def _pack_body(*refs):
    ins = refs[:NUM_FIELDS]
    outs = refs[NUM_FIELDS:]
    eye = jnp.eye(128, dtype=jnp.float32)
    for t, o in zip(ins, outs):
        x = t[...]
        for h in range(PACK_BN // 1024):
            folded = jnp.concatenate(
                [x[:, 128 * (8 * h + k):128 * (8 * h + k + 1)]
                 for k in range(8)], axis=0)
            o[128 * h:128 * (h + 1), :] = jax.lax.dot_general(
                folded, eye, (((0,), (0,)), ((), ())),
                preferred_element_type=jnp.float32)


def _pack(tabTs):
    return pl.pallas_call(
        _pack_body,
        grid=(PACK_GRID,),
        in_specs=[pl.BlockSpec((EMB, PACK_BN), lambda i: (0, i))] * NUM_FIELDS,
        out_specs=[pl.BlockSpec((PACK_BN // 8, 8 * EMB), lambda i: (i, 0))] * NUM_FIELDS,
        out_shape=[jax.ShapeDtypeStruct((VOCAB_P // 8, 8 * EMB), jnp.float32)] * NUM_FIELDS,
    )(*tabTs)


def _gather_body(*refs):
    tables = refs[:NUM_FIELDS]
    idxs = refs[NUM_FIELDS:2 * NUM_FIELDS]
    out = refs[2 * NUM_FIELDS]
    idx_v, rows_v, sem = refs[2 * NUM_FIELDS + 1:]
    wid = lax.axis_index("s") * 2 + lax.axis_index("c")
    base = wid * BPW
    for f in range(NUM_FIELDS):
        pltpu.sync_copy(idxs[f].at[pl.ds(base, BPW)], idx_v)
        pltpu.async_copy(tables[f].at[idx_v], rows_v, sem).wait()
        pltpu.sync_copy(rows_v, out.at[pl.ds(base, BPW), pl.ds(f * EMB, EMB)])


_gather = pl.kernel(
    _gather_body,
    out_type=jax.ShapeDtypeStruct((BATCH, NUM_FIELDS * EMB), jnp.float32),
    mesh=plsc.VectorSubcoreMesh(core_axis_name="c", subcore_axis_name="s"),
    scratch_types=[
        pltpu.VMEM((BPW,), jnp.int32),
        pltpu.VMEM((BPW, EMB), jnp.float32),
        pltpu.SemaphoreType.DMA,
    ],
    compiler_params=pltpu.CompilerParams(use_tc_tiling_on_sc=False),
)


def _mlp_body(x_ref, w1_ref, b1_ref, w2_ref, b2_ref, o_ref):
    h = jnp.dot(x_ref[...], w1_ref[...], preferred_element_type=jnp.float32)
    h = jnp.maximum(h + b1_ref[...], 0.0)
    o = jnp.dot(h, w2_ref[...], preferred_element_type=jnp.float32)
    o_ref[...] = jnp.maximum(o + b2_ref[...], 0.0)


def _mlp(x, w1, b1, w2, b2, bb=2048):
    d = NUM_FIELDS * EMB
    return pl.pallas_call(
        _mlp_body,
        grid=(BATCH // bb,),
        in_specs=[
            pl.BlockSpec((bb, d), lambda i: (i, 0)),
            pl.BlockSpec((d, H1), lambda i: (0, 0)),
            pl.BlockSpec((1, H1), lambda i: (0, 0)),
            pl.BlockSpec((H1, H2), lambda i: (0, 0)),
            pl.BlockSpec((1, H2), lambda i: (0, 0)),
        ],
        out_specs=pl.BlockSpec((bb, H2), lambda i: (i, 0)),
        out_shape=jax.ShapeDtypeStruct((BATCH, H2), jnp.float32),
    )(x, w1, b1, w2, b2)


def kernel(feat_0, feat_1, feat_2, feat_3, feat_4, feat_5, feat_6, feat_7,
           feat_8, feat_9, feat_10, feat_11, feat_12, feat_13, feat_14,
           feat_15, feat_16, feat_17, feat_18, feat_19, feat_20, feat_21,
           feat_22, feat_23, feat_24, feat_25,
           E_0, E_1, E_2, E_3, E_4, E_5, E_6, E_7, E_8, E_9, E_10, E_11,
           E_12, E_13, E_14, E_15, E_16, E_17, E_18, E_19, E_20, E_21,
           E_22, E_23, E_24, E_25,
           W1, b1, W2, b2):
    feats = [feat_0, feat_1, feat_2, feat_3, feat_4, feat_5, feat_6, feat_7,
             feat_8, feat_9, feat_10, feat_11, feat_12, feat_13, feat_14,
             feat_15, feat_16, feat_17, feat_18, feat_19, feat_20, feat_21,
             feat_22, feat_23, feat_24, feat_25]
    tables = [E_0, E_1, E_2, E_3, E_4, E_5, E_6, E_7, E_8, E_9, E_10, E_11,
              E_12, E_13, E_14, E_15, E_16, E_17, E_18, E_19, E_20, E_21,
              E_22, E_23, E_24, E_25]
    feats = [jnp.asarray(f, jnp.int32) for f in feats]
    feats = [
        (f & ~1023) + ((f & 127) << 3) + ((f >> 7) & 7) for f in feats
    ]
    packed = _pack([t.T for t in tables])
    tables = [p.reshape(VOCAB_P, EMB) for p in packed]
    x = _gather(*tables, *feats)
    return _mlp(x, W1, b1.reshape(1, H1), W2, b2.reshape(1, H2))


# speedup vs baseline: 5.4222x; 1.0619x over previous
import jax
import jax.numpy as jnp
from jax import lax
from jax.experimental import pallas as pl
from jax.experimental.pallas import tpu as pltpu
from jax.experimental.pallas import tpu_sc as plsc

NUM_FIELDS = 26
NF_G = 13
BATCH = 16384
VOCAB = 100000
VOCAB_P = 100352
EMB = 16
H1 = 128
H2 = 64
NW = 32
BPW = BATCH // NW

PACK_BN = 2048
PACK_GRID = -(-VOCAB_P // PACK_BN)


def _pack_body(*refs):
    ins = refs[:NF_G]
    outs = refs[NF_G:]
    eye = jnp.eye(128, dtype=jnp.float32)
    for t, o in zip(ins, outs):
        x = t[...]
        for h in range(PACK_BN // 1024):
            folded = jnp.concatenate(
                [x[:, 128 * (8 * h + k):128 * (8 * h + k + 1)]
                 for k in range(8)], axis=0)
            o[128 * h:128 * (h + 1), :] = jax.lax.dot_general(
                folded, eye, (((0,), (0,)), ((), ())),
                preferred_element_type=jnp.float32)


def _pack(tabTs):
    return pl.pallas_call(
        _pack_body,
        grid=(PACK_GRID,),
        in_specs=[pl.BlockSpec((EMB, PACK_BN), lambda i: (0, i))] * NF_G,
        out_specs=[pl.BlockSpec((PACK_BN // 8, 8 * EMB), lambda i: (i, 0))] * NF_G,
        out_shape=[jax.ShapeDtypeStruct((VOCAB_P // 8, 8 * EMB), jnp.float32)] * NF_G,
    )(*tabTs)


def _gather_body(*refs):
    tables = refs[:NF_G]
    idxs = refs[NF_G:2 * NF_G]
    out = refs[2 * NF_G]
    idx_v, rows_v, sem = refs[2 * NF_G + 1:]
    wid = lax.axis_index("s") * 2 + lax.axis_index("c")
    base = wid * BPW
    for f in range(NF_G):
        pltpu.sync_copy(idxs[f].at[pl.ds(base, BPW)], idx_v)
        pltpu.async_copy(tables[f].at[idx_v], rows_v, sem).wait()
        pltpu.sync_copy(rows_v, out.at[pl.ds(base, BPW), pl.ds(f * EMB, EMB)])


_gather = pl.kernel(
    _gather_body,
    out_type=jax.ShapeDtypeStruct((BATCH, NF_G * EMB), jnp.float32),
    mesh=plsc.VectorSubcoreMesh(core_axis_name="c", subcore_axis_name="s"),
    scratch_types=[
        pltpu.VMEM((BPW,), jnp.int32),
        pltpu.VMEM((BPW, EMB), jnp.float32),
        pltpu.SemaphoreType.DMA,
    ],
    compiler_params=pltpu.CompilerParams(use_tc_tiling_on_sc=False),
)


def _mlp_body(x1_ref, x2_ref, w1a_ref, w1b_ref, b1_ref, w2_ref, b2_ref, o_ref):
    h = jnp.dot(x1_ref[...], w1a_ref[...], preferred_element_type=jnp.float32)
    h = h + jnp.dot(x2_ref[...], w1b_ref[...], preferred_element_type=jnp.float32)
    h = jnp.maximum(h + b1_ref[...], 0.0)
    o = jnp.dot(h, w2_ref[...], preferred_element_type=jnp.float32)
    o_ref[...] = jnp.maximum(o + b2_ref[...], 0.0)


def _mlp(x1, x2, w1a, w1b, b1, w2, b2, bb=2048):
    d = NF_G * EMB
    return pl.pallas_call(
        _mlp_body,
        grid=(BATCH // bb,),
        in_specs=[
            pl.BlockSpec((bb, d), lambda i: (i, 0)),
            pl.BlockSpec((bb, d), lambda i: (i, 0)),
            pl.BlockSpec((d, H1), lambda i: (0, 0)),
            pl.BlockSpec((d, H1), lambda i: (0, 0)),
            pl.BlockSpec((1, H1), lambda i: (0, 0)),
            pl.BlockSpec((H1, H2), lambda i: (0, 0)),
            pl.BlockSpec((1, H2), lambda i: (0, 0)),
        ],
        out_specs=pl.BlockSpec((bb, H2), lambda i: (i, 0)),
        out_shape=jax.ShapeDtypeStruct((BATCH, H2), jnp.float32),
    )(x1, x2, w1a, w1b, b1, w2, b2)


def kernel(feat_0, feat_1, feat_2, feat_3, feat_4, feat_5, feat_6, feat_7,
           feat_8, feat_9, feat_10, feat_11, feat_12, feat_13, feat_14,
           feat_15, feat_16, feat_17, feat_18, feat_19, feat_20, feat_21,
           feat_22, feat_23, feat_24, feat_25,
           E_0, E_1, E_2, E_3, E_4, E_5, E_6, E_7, E_8, E_9, E_10, E_11,
           E_12, E_13, E_14, E_15, E_16, E_17, E_18, E_19, E_20, E_21,
           E_22, E_23, E_24, E_25,
           W1, b1, W2, b2):
    feats = [feat_0, feat_1, feat_2, feat_3, feat_4, feat_5, feat_6, feat_7,
             feat_8, feat_9, feat_10, feat_11, feat_12, feat_13, feat_14,
             feat_15, feat_16, feat_17, feat_18, feat_19, feat_20, feat_21,
             feat_22, feat_23, feat_24, feat_25]
    tables = [E_0, E_1, E_2, E_3, E_4, E_5, E_6, E_7, E_8, E_9, E_10, E_11,
              E_12, E_13, E_14, E_15, E_16, E_17, E_18, E_19, E_20, E_21,
              E_22, E_23, E_24, E_25]
    feats = [jnp.asarray(f, jnp.int32) for f in feats]
    feats = [
        (f & ~1023) + ((f & 127) << 3) + ((f >> 7) & 7) for f in feats
    ]
    halves = []
    for g in range(2):
        packed = _pack([t.T for t in tables[g * NF_G:(g + 1) * NF_G]])
        ptabs = [p.reshape(VOCAB_P, EMB) for p in packed]
        halves.append(_gather(*ptabs, *feats[g * NF_G:(g + 1) * NF_G]))
    d = NF_G * EMB
    return _mlp(halves[0], halves[1], W1[:d], W1[d:],
                b1.reshape(1, H1), W2, b2.reshape(1, H2))
